# baseline (speedup 1.0000x reference)
import functools
import math

import jax
import jax.numpy as jnp
from jax import lax
from jax.experimental import pallas as pl
from jax.experimental.pallas import tpu as pltpu
from jax.experimental.pallas import tpu_sc as plsc

D_MODEL = 64
SCALE = math.sqrt(D_MODEL)

NUM_CORES = 2
NUM_SUBCORES = 16
NUM_WORKERS = NUM_CORES * NUM_SUBCORES

BATCH_PER_CHUNK = 16
LANES = 16


def kernel(x, weight):
    batch, hist = x.shape
    vocab, d = weight.shape
    n = batch * hist
    idx = x.reshape(n).astype(jnp.int32)

    b_per_w = batch // NUM_WORKERS
    rows_per_w = b_per_w * hist
    chunk_rows = BATCH_PER_CHUNK * hist
    num_chunks = b_per_w // BATCH_PER_CHUNK

    def emb_kernel(table_hbm, idx_hbm, out_hbm,
                   idx_v, rows_a, rows_b, gsem_a, gsem_b, osem_a, osem_b):
        wid = lax.axis_index("s") * NUM_CORES + lax.axis_index("c")
        row_base = wid * rows_per_w
        batch_base = wid * b_per_w

        pltpu.sync_copy(idx_hbm.at[pl.ds(row_base, rows_per_w)], idx_v)

        def gather_start(g, buf, sem):
            return pltpu.async_copy(
                table_hbm.at[idx_v.at[pl.ds(g * chunk_rows, chunk_rows)]],
                buf, sem)

        def gather_wait(buf, sem):
            pltpu.make_async_copy(
                table_hbm.at[idx_v.at[pl.ds(0, chunk_rows)]], buf, sem).wait()

        def scale_chunk(buf):
            @plsc.parallel_loop(0, chunk_rows, step=1, unroll=4)
            def _(r):
                for c in range(D_MODEL // LANES):
                    sl = pl.ds(c * LANES, LANES)
                    buf[r, sl] = buf[r, sl] * SCALE

        def store_start(g, buf, sem):
            b0 = batch_base + g * BATCH_PER_CHUNK
            for k in range(BATCH_PER_CHUNK):
                pltpu.async_copy(
                    buf.at[pl.ds(k * hist, hist)], out_hbm.at[b0 + k], sem)

        def store_wait(buf, sem):
            for k in range(BATCH_PER_CHUNK):
                pltpu.make_async_copy(
                    buf.at[pl.ds(k * hist, hist)], out_hbm.at[0], sem).wait()

        m = num_chunks

        gather_start(0, rows_a, gsem_a)
        gather_start(1, rows_b, gsem_b)
        gather_wait(rows_a, gsem_a)
        scale_chunk(rows_a)
        store_start(0, rows_a, osem_a)

        def pair_body(p, _):
            g1 = 1 + 2 * p
            store_wait(rows_a, osem_a)
            gather_start(g1 + 1, rows_a, gsem_a)
            gather_wait(rows_b, gsem_b)
            scale_chunk(rows_b)
            store_start(g1, rows_b, osem_b)
            store_wait(rows_b, osem_b)
            gather_start(g1 + 2, rows_b, gsem_b)
            gather_wait(rows_a, gsem_a)
            scale_chunk(rows_a)
            store_start(g1 + 1, rows_a, osem_a)
            return 0

        lax.fori_loop(0, (m - 2) // 2, pair_body, 0)

        gather_wait(rows_b, gsem_b)
        scale_chunk(rows_b)
        store_start(m - 1, rows_b, osem_b)
        store_wait(rows_a, osem_a)
        store_wait(rows_b, osem_b)

    mesh = plsc.VectorSubcoreMesh(core_axis_name="c", subcore_axis_name="s")
    run = pl.kernel(
        emb_kernel,
        out_type=jax.ShapeDtypeStruct((batch, hist, d), jnp.float32),
        mesh=mesh,
        scratch_types=[
            pltpu.VMEM((rows_per_w,), jnp.int32),
            pltpu.VMEM((chunk_rows, d), jnp.float32),
            pltpu.VMEM((chunk_rows, d), jnp.float32),
            pltpu.SemaphoreType.DMA,
            pltpu.SemaphoreType.DMA,
            pltpu.SemaphoreType.DMA,
            pltpu.SemaphoreType.DMA,
        ],
        compiler_params=pltpu.CompilerParams(use_tc_tiling_on_sc=False),
    )
    return run(weight, idx)


# speedup vs baseline: 1.1199x; 1.1199x over previous
import functools
import math

import jax
import jax.numpy as jnp
from jax import lax
from jax.experimental import pallas as pl
from jax.experimental.pallas import tpu as pltpu
from jax.experimental.pallas import tpu_sc as plsc

D_MODEL = 64
SCALE = math.sqrt(D_MODEL)

NUM_CORES = 2
NUM_SUBCORES = 16
NUM_WORKERS = NUM_CORES * NUM_SUBCORES

BATCH_PER_CHUNK = 16
LANES = 16


def kernel(x, weight):
    batch, hist = x.shape
    vocab, d = weight.shape
    n = batch * hist
    idx = x.reshape(n).astype(jnp.int32)

    b_per_w = batch // NUM_WORKERS
    rows_per_w = b_per_w * hist
    chunk_rows = BATCH_PER_CHUNK * hist
    num_chunks = b_per_w // BATCH_PER_CHUNK

    def emb_kernel(table_hbm, idx_hbm, out_hbm,
                   idx_v, rows_a, rows_b, gsem_a, gsem_b, osem_a, osem_b):
        wid = lax.axis_index("s") * NUM_CORES + lax.axis_index("c")
        row_base = wid * rows_per_w
        batch_base = wid * b_per_w

        pltpu.sync_copy(idx_hbm.at[pl.ds(row_base, rows_per_w)], idx_v)

        def gather_start(g, buf, sem):
            return pltpu.async_copy(
                table_hbm.at[idx_v.at[pl.ds(g * chunk_rows, chunk_rows)]],
                buf, sem)

        def gather_wait(buf, sem):
            pltpu.make_async_copy(
                table_hbm.at[idx_v.at[pl.ds(0, chunk_rows)]], buf, sem).wait()

        def scale_chunk(buf):
            @plsc.parallel_loop(0, chunk_rows, step=1, unroll=4)
            def _(r):
                for c in range(D_MODEL // LANES):
                    sl = pl.ds(c * LANES, LANES)
                    buf[r, sl] = buf[r, sl] * SCALE

        def store_start(g, buf, sem):
            b0 = batch_base + g * BATCH_PER_CHUNK
            for k in range(BATCH_PER_CHUNK):
                pltpu.async_copy(
                    buf.at[pl.ds(k * hist, hist)],
                    out_hbm.at[b0 + k, pl.ds(0, hist), pl.ds(0, d)], sem)

        def store_wait(buf, sem):
            for k in range(BATCH_PER_CHUNK):
                pltpu.make_async_copy(
                    buf.at[pl.ds(k * hist, hist)],
                    out_hbm.at[0, pl.ds(0, hist), pl.ds(0, d)], sem).wait()

        m = num_chunks

        gather_start(0, rows_a, gsem_a)
        gather_start(1, rows_b, gsem_b)
        gather_wait(rows_a, gsem_a)
        scale_chunk(rows_a)
        store_start(0, rows_a, osem_a)

        def pair_body(p, _):
            g1 = 1 + 2 * p
            store_wait(rows_a, osem_a)
            gather_start(g1 + 1, rows_a, gsem_a)
            gather_wait(rows_b, gsem_b)
            scale_chunk(rows_b)
            store_start(g1, rows_b, osem_b)
            store_wait(rows_b, osem_b)
            gather_start(g1 + 2, rows_b, gsem_b)
            gather_wait(rows_a, gsem_a)
            scale_chunk(rows_a)
            store_start(g1 + 1, rows_a, osem_a)
            return 0

        lax.fori_loop(0, (m - 2) // 2, pair_body, 0)

        gather_wait(rows_b, gsem_b)
        scale_chunk(rows_b)
        store_start(m - 1, rows_b, osem_b)
        store_wait(rows_a, osem_a)
        store_wait(rows_b, osem_b)

    hist_pad = (hist + 7) // 8 * 8
    d_pad = 128
    mesh = plsc.VectorSubcoreMesh(core_axis_name="c", subcore_axis_name="s")
    run = pl.kernel(
        emb_kernel,
        out_type=jax.ShapeDtypeStruct((batch, hist_pad, d_pad), jnp.float32),
        mesh=mesh,
        scratch_types=[
            pltpu.VMEM((rows_per_w,), jnp.int32),
            pltpu.VMEM((chunk_rows, d), jnp.float32),
            pltpu.VMEM((chunk_rows, d), jnp.float32),
            pltpu.SemaphoreType.DMA,
            pltpu.SemaphoreType.DMA,
            pltpu.SemaphoreType.DMA,
            pltpu.SemaphoreType.DMA,
        ],
        compiler_params=pltpu.CompilerParams(use_tc_tiling_on_sc=False),
    )
    out = run(weight, idx)
    return out[:, :hist, :d]


# speedup vs baseline: 1.1216x; 1.0015x over previous
import functools
import math

import jax
import jax.numpy as jnp
from jax import lax
from jax.experimental import pallas as pl
from jax.experimental.pallas import tpu as pltpu
from jax.experimental.pallas import tpu_sc as plsc

D_MODEL = 64
SCALE = math.sqrt(D_MODEL)

NUM_CORES = 2
NUM_SUBCORES = 16
NUM_WORKERS = NUM_CORES * NUM_SUBCORES

BATCH_PER_CHUNK = 16
LANES = 16


def _flatten_idx(x):
    batch, hist = x.shape
    n = batch * hist
    b_per_w = batch // NUM_WORKERS
    GROUP = 4
    flat_len = GROUP * hist

    def body(x_hbm, out_hbm, xv, fv, sem):
        wid = lax.axis_index("s") * NUM_CORES + lax.axis_index("c")
        b0 = wid * b_per_w
        pltpu.sync_copy(x_hbm.at[pl.ds(b0, b_per_w)], xv)

        def group_body(g, _):
            for r in range(GROUP):
                row = g * GROUP + r
                for off in (0, 16, 32, hist - 16):
                    fv[pl.ds(r * hist + off, 16)] = xv[row, pl.ds(off, 16)]
            pltpu.async_copy(
                fv, out_hbm.at[pl.ds((b0 + g * GROUP) * hist, flat_len)],
                sem).wait()
            return 0

        lax.fori_loop(0, b_per_w // GROUP, group_body, 0)

    mesh = plsc.VectorSubcoreMesh(core_axis_name="c", subcore_axis_name="s")
    run = pl.kernel(
        body,
        out_type=jax.ShapeDtypeStruct((n,), jnp.int32),
        mesh=mesh,
        scratch_types=[
            pltpu.VMEM((b_per_w, hist), jnp.int32),
            pltpu.VMEM((flat_len,), jnp.int32),
            pltpu.SemaphoreType.DMA,
        ],
        compiler_params=pltpu.CompilerParams(use_tc_tiling_on_sc=True),
    )
    return run(x)


def kernel(x, weight):
    batch, hist = x.shape
    vocab, d = weight.shape
    n = batch * hist
    idx = _flatten_idx(x)

    b_per_w = batch // NUM_WORKERS
    rows_per_w = b_per_w * hist
    chunk_rows = BATCH_PER_CHUNK * hist
    num_chunks = b_per_w // BATCH_PER_CHUNK

    def emb_kernel(table_hbm, idx_hbm, out_hbm,
                   idx_v, rows_a, rows_b, gsem_a, gsem_b, osem_a, osem_b):
        wid = lax.axis_index("s") * NUM_CORES + lax.axis_index("c")
        row_base = wid * rows_per_w
        batch_base = wid * b_per_w

        pltpu.sync_copy(idx_hbm.at[pl.ds(row_base, rows_per_w)], idx_v)

        def gather_start(g, buf, sem):
            return pltpu.async_copy(
                table_hbm.at[idx_v.at[pl.ds(g * chunk_rows, chunk_rows)]],
                buf, sem)

        def gather_wait(buf, sem):
            pltpu.make_async_copy(
                table_hbm.at[idx_v.at[pl.ds(0, chunk_rows)]], buf, sem).wait()

        def scale_chunk(buf):
            @plsc.parallel_loop(0, chunk_rows, step=1, unroll=4)
            def _(r):
                for c in range(D_MODEL // LANES):
                    sl = pl.ds(c * LANES, LANES)
                    buf[r, sl] = buf[r, sl] * SCALE

        def store_start(g, buf, sem):
            b0 = batch_base + g * BATCH_PER_CHUNK
            for k in range(BATCH_PER_CHUNK):
                pltpu.async_copy(
                    buf.at[pl.ds(k * hist, hist)],
                    out_hbm.at[b0 + k, pl.ds(0, hist), pl.ds(0, d)], sem)

        def store_wait(buf, sem):
            for k in range(BATCH_PER_CHUNK):
                pltpu.make_async_copy(
                    buf.at[pl.ds(k * hist, hist)],
                    out_hbm.at[0, pl.ds(0, hist), pl.ds(0, d)], sem).wait()

        m = num_chunks

        gather_start(0, rows_a, gsem_a)
        gather_start(1, rows_b, gsem_b)
        gather_wait(rows_a, gsem_a)
        scale_chunk(rows_a)
        store_start(0, rows_a, osem_a)

        def pair_body(p, _):
            g1 = 1 + 2 * p
            store_wait(rows_a, osem_a)
            gather_start(g1 + 1, rows_a, gsem_a)
            gather_wait(rows_b, gsem_b)
            scale_chunk(rows_b)
            store_start(g1, rows_b, osem_b)
            store_wait(rows_b, osem_b)
            gather_start(g1 + 2, rows_b, gsem_b)
            gather_wait(rows_a, gsem_a)
            scale_chunk(rows_a)
            store_start(g1 + 1, rows_a, osem_a)
            return 0

        lax.fori_loop(0, (m - 2) // 2, pair_body, 0)

        gather_wait(rows_b, gsem_b)
        scale_chunk(rows_b)
        store_start(m - 1, rows_b, osem_b)
        store_wait(rows_a, osem_a)
        store_wait(rows_b, osem_b)

    hist_pad = (hist + 7) // 8 * 8
    d_pad = 128
    mesh = plsc.VectorSubcoreMesh(core_axis_name="c", subcore_axis_name="s")
    run = pl.kernel(
        emb_kernel,
        out_type=jax.ShapeDtypeStruct((batch, hist_pad, d_pad), jnp.float32),
        mesh=mesh,
        scratch_types=[
            pltpu.VMEM((rows_per_w,), jnp.int32),
            pltpu.VMEM((chunk_rows, d), jnp.float32),
            pltpu.VMEM((chunk_rows, d), jnp.float32),
            pltpu.SemaphoreType.DMA,
            pltpu.SemaphoreType.DMA,
            pltpu.SemaphoreType.DMA,
            pltpu.SemaphoreType.DMA,
        ],
        compiler_params=pltpu.CompilerParams(use_tc_tiling_on_sc=False),
    )
    out = run(weight, idx)
    return out[:, :hist, :d]
